# baseline (speedup 1.0000x reference)
import dataclasses
import functools

import jax
import jax.numpy as jnp
from jax import lax
from jax.experimental import pallas as pl
from jax.experimental.pallas import tpu as pltpu
from jax.experimental.pallas import tpu_sc as plsc

N_NODES = 10242
BLK = 256
NPAD = ((N_NODES + BLK - 1) // BLK) * BLK
RNB = 75
NUMIDX = RNB * NPAD
GW = 128
NGATH = NUMIDX // GW
EPS = 1e-5
NC, NS = 2, 16
NW = NC * NS
GPW = NGATH // NW
GREM = NGATH - GPW * NW
MAXG = -(-(GPW + GREM) // 8) * 8
NGPAD = (NW - 1) * GPW + MAXG
TDT = jnp.float32


def _sc_gather(table, idx2d):
    mesh = plsc.VectorSubcoreMesh(core_axis_name="c", subcore_axis_name="s")
    nbuf = 6

    @functools.partial(
        pl.kernel,
        mesh=mesh,
        out_type=jax.ShapeDtypeStruct((NUMIDX, 128), table.dtype),
        scratch_types=[pltpu.VMEM((MAXG, GW), jnp.int32)]
        + [pltpu.VMEM((GW, 128), table.dtype)] * nbuf
        + [pltpu.SemaphoreType.DMA] * nbuf,
    )
    def gk(table_hbm, idx_hbm, out_hbm, idx_s, *bufs_sems):
        bufs, sems = bufs_sems[:nbuf], bufs_sems[nbuf:]
        wid = lax.axis_index("s") * NC + lax.axis_index("c")
        g0 = wid * GPW
        ng = lax.select(wid == NW - 1, GPW + GREM, GPW)
        pltpu.sync_copy(idx_hbm.at[pl.ds(g0, MAXG)], idx_s)

        @pl.loop(0, ng // nbuf)
        def _(it):
            c = it * nbuf
            cps = [
                pltpu.async_copy(table_hbm.at[idx_s.at[c + b]], bufs[b],
                                 sems[b])
                for b in range(nbuf)
            ]
            for b in range(nbuf):
                cps[b].wait()
                pltpu.sync_copy(bufs[b],
                                out_hbm.at[pl.ds((g0 + c + b) * GW, GW)])

    return gk(table, idx2d)


XFL = 30728
NPW = NPAD // NW
L0C = 8


def _sc_interp0(xflat, idx3, w3):
    mesh = plsc.VectorSubcoreMesh(core_axis_name="c", subcore_axis_name="s")
    nch = NPW // L0C
    cp = pltpu.CompilerParams()
    if "needs_layout_passes" in pltpu.CompilerParams.__dataclass_fields__:
        cp = dataclasses.replace(cp, needs_layout_passes=False)

    @functools.partial(
        pl.kernel,
        mesh=mesh,
        compiler_params=cp,
        out_type=jax.ShapeDtypeStruct((NPAD, 128), jnp.float32),
        scratch_types=[
            pltpu.VMEM((1, XFL), jnp.float32),
            pltpu.VMEM((L0C, 256), jnp.int32),
            pltpu.VMEM((L0C, 256), jnp.float32),
            pltpu.VMEM((L0C, 128), jnp.float32),
            pltpu.SemaphoreType.DMA,
        ],
    )
    def ik(x_hbm, i_hbm, w_hbm, o_hbm, x_v, i_v, w_v, o_v, sem):
        wid = lax.axis_index("s") * NC + lax.axis_index("c")
        n0 = wid * NPW
        pltpu.sync_copy(x_hbm, x_v)

        @pl.loop(0, nch)
        def _(ci):
            base = n0 + ci * L0C
            pltpu.sync_copy(i_hbm.at[pl.ds(base, L0C)], i_v)
            pltpu.sync_copy(w_hbm.at[pl.ds(base, L0C)], w_v)
            for n in range(L0C):
                for v in range(5):
                    acc = None
                    for j in range(3):
                        o = j * 80 + v * 16
                        ivec = i_v[n, pl.ds(o, 16)]
                        term = (w_v[n, pl.ds(o, 16)]
                                * plsc.load_gather(x_v.at[0], [ivec]))
                        acc = term if acc is None else acc + term
                    o_v[n, pl.ds(v * 16, 16)] = acc
                o_v[n, pl.ds(80, 16)] = jnp.zeros((16,), jnp.float32)
                o_v[n, pl.ds(96, 16)] = jnp.zeros((16,), jnp.float32)
                o_v[n, pl.ds(112, 16)] = jnp.zeros((16,), jnp.float32)
            pltpu.sync_copy(o_v, o_hbm.at[pl.ds(base, L0C)])

    return ik(xflat, idx3, w3)


def _l0_streams(neigh_indices, neigh_weights):
    idxr = neigh_indices.reshape(N_NODES, 25, 3)
    nwr = neigh_weights
    cc = jnp.tile(jnp.arange(3, dtype=jnp.int32), 25)
    parts_i, parts_w = [], []
    for j in range(3):
        a = jnp.repeat(idxr[:, :, j], 3, axis=1) * 3 + cc
        w = jnp.repeat(nwr[:, :, j], 3, axis=1)
        parts_i.append(jnp.pad(a, ((0, 0), (0, 5))))
        parts_w.append(jnp.pad(w, ((0, 0), (0, 5))))
    idx3 = jnp.pad(jnp.concatenate(parts_i, axis=1),
                   ((0, NPAD - N_NODES), (0, 16)))
    w3 = jnp.pad(jnp.concatenate(parts_w, axis=1),
                 ((0, NPAD - N_NODES), (0, 16)))
    return idx3, w3


def _matmul0(interp0, we, b, o, interpret=False):
    nblk = NPAD // BLK
    kdim = interp0.shape[1]

    def body(a_ref, we_ref, b_ref, y_ref, sm_ref, sq_ref):
        i = pl.program_id(0)
        y = jnp.dot(a_ref[...], we_ref[...],
                    preferred_element_type=jnp.float32) + b_ref[0]
        y_ref[...] = y
        rid = i * BLK + lax.broadcasted_iota(jnp.int32, (BLK, 1), 0)
        ym = jnp.where(rid < N_NODES, y, 0.0)
        ps = jnp.sum(ym, axis=0, keepdims=True)
        ps2 = jnp.sum(ym * ym, axis=0, keepdims=True)

        @pl.when(i == 0)
        def _():
            sm_ref[...] = ps
            sq_ref[...] = ps2

        @pl.when(i > 0)
        def _():
            sm_ref[...] = sm_ref[...] + ps
            sq_ref[...] = sq_ref[...] + ps2

    return pl.pallas_call(
        body,
        grid=(nblk,),
        in_specs=[
            pl.BlockSpec((BLK, kdim), lambda i: (i, 0)),
            pl.BlockSpec((kdim, o), lambda i: (0, 0)),
            pl.BlockSpec((1, o), lambda i: (0, 0)),
        ],
        out_specs=[
            pl.BlockSpec((BLK, o), lambda i: (i, 0)),
            pl.BlockSpec((1, o), lambda i: (0, 0)),
            pl.BlockSpec((1, o), lambda i: (0, 0)),
        ],
        out_shape=[
            jax.ShapeDtypeStruct((NPAD, o), jnp.float32),
            jax.ShapeDtypeStruct((1, o), jnp.float32),
            jax.ShapeDtypeStruct((1, o), jnp.float32),
        ],
        interpret=interpret,
    )(interp0, we, b)


CHN = 4
NCH = NPAD // CHN
CHT = CHN * RNB
CHTP = CHT + 4
CPW = NCH // NW


def _sc_conv_interp(table, idxs, ws, cin):
    mesh = plsc.VectorSubcoreMesh(core_axis_name="c", subcore_axis_name="s")
    cp = pltpu.CompilerParams()
    if "needs_layout_passes" in pltpu.CompilerParams.__dataclass_fields__:
        cp = dataclasses.replace(cp, needs_layout_passes=False)
    kw = 25 * cin
    obuf = 8 * kw
    nv = cin // 16

    @functools.partial(
        pl.kernel,
        mesh=mesh,
        compiler_params=cp,
        out_type=jax.ShapeDtypeStruct((NPAD * kw,), jnp.float32),
        scratch_types=[
            pltpu.VMEM((CHTP,), jnp.int32),
            pltpu.VMEM((CHTP,), jnp.int32),
            pltpu.VMEM((CHTP, 128), jnp.float32),
            pltpu.VMEM((CHTP, 128), jnp.float32),
            pltpu.VMEM((CHTP * 16,), jnp.float32),
            pltpu.VMEM((obuf,), jnp.float32),
            pltpu.SemaphoreType.DMA,
            pltpu.SemaphoreType.DMA,
        ],
    )
    def ck(t_hbm, i_hbm, w_hbm, o_hbm, ia_v, ib_v, ta_v, tb_v, w_v, o_v,
           sa, sb):
        wid = lax.axis_index("s") * NC + lax.axis_index("c")
        c0 = wid * CPW

        def fetch(c, idx_v, taps_v, sem):
            pltpu.sync_copy(i_hbm.at[pl.ds(c * CHTP, CHTP)], idx_v)
            pltpu.make_async_copy(t_hbm.at[idx_v], taps_v, sem).start()

        def wait(idx_v, taps_v, sem):
            pltpu.make_async_copy(t_hbm.at[idx_v], taps_v, sem).wait()

        def compute(c, taps_v, half):
            pltpu.sync_copy(w_hbm.at[pl.ds(c * CHTP * 16, CHTP * 16)], w_v)
            for n in range(CHN):
                @pl.loop(0, 25)
                def _(k):
                    r0 = n * RNB + 3 * k
                    ob = (half * CHN + n) * kw + k * cin
                    for v in range(nv):
                        acc = None
                        for j in range(3):
                            wv = w_v[pl.ds((r0 + j) * 16, 16)]
                            term = wv * taps_v[r0 + j, pl.ds(16 * v, 16)]
                            acc = term if acc is None else acc + term
                        o_v[pl.ds(ob + 16 * v, 16)] = acc

        fetch(c0, ia_v, ta_v, sa)

        @pl.loop(0, CPW // 2)
        def _(p):
            ca = c0 + 2 * p
            fetch(ca + 1, ib_v, tb_v, sb)
            wait(ia_v, ta_v, sa)
            compute(ca, ta_v, 0)
            fetch(ca + 2, ia_v, ta_v, sa)
            wait(ib_v, tb_v, sb)
            compute(ca + 1, tb_v, 1)
            pltpu.sync_copy(o_v, o_hbm.at[pl.ds((ca * CHN) * kw, obuf)])

        wait(ia_v, ta_v, sa)

    return ck(table, idxs, ws)


def _tap_streams(neigh_indices, neigh_weights):
    def prep(a, dt, rep):
        a = jnp.pad(a.reshape(N_NODES, RNB).astype(dt),
                    ((0, NPAD - N_NODES), (0, 0)))
        if rep > 1:
            a = jnp.repeat(a.reshape(-1, 1), rep, axis=1)
        a = jnp.pad(a.reshape(NCH, CHT * rep),
                    ((0, 8), (0, (CHTP - CHT) * rep)))
        return a.reshape(-1)

    return (prep(neigh_indices, jnp.int32, 1),
            prep(neigh_weights, jnp.float32, 16))


def _expand_w(w, ci, cp):
    o = w.shape[0]
    wr = w.reshape(o, 25, ci)
    if cp != ci:
        wr = jnp.pad(wr, ((0, 0), (0, 0), (0, cp - ci)))
    return wr.transpose(1, 2, 0).reshape(25 * cp, o)


def _conv(g, w_tn, we, b, o, cp, interpret=False):
    nblk = NPAD // BLK

    def body(g_ref, w_ref, we_ref, b_ref, y_ref, sm_ref, sq_ref, interp_ref):
        i = pl.program_id(0)
        for k in range(25):
            acc = None
            for j in range(3):
                r = 3 * k + j
                gr = g_ref[r, :, :cp].astype(jnp.float32)
                term = gr * w_ref[:, r : r + 1]
                acc = term if acc is None else acc + term
            interp_ref[:, k * cp : (k + 1) * cp] = acc
        y = jnp.dot(interp_ref[:], we_ref[:],
                    preferred_element_type=jnp.float32) + b_ref[0]
        y_ref[:] = y
        rid = i * BLK + lax.broadcasted_iota(jnp.int32, (BLK, 1), 0)
        ym = jnp.where(rid < N_NODES, y, 0.0)
        ps = jnp.sum(ym, axis=0, keepdims=True)
        ps2 = jnp.sum(ym * ym, axis=0, keepdims=True)

        @pl.when(i == 0)
        def _():
            sm_ref[:] = ps
            sq_ref[:] = ps2

        @pl.when(i > 0)
        def _():
            sm_ref[:] = sm_ref[:] + ps
            sq_ref[:] = sq_ref[:] + ps2

    return pl.pallas_call(
        body,
        grid=(nblk,),
        in_specs=[
            pl.BlockSpec((RNB, BLK, 128), lambda i: (0, i, 0)),
            pl.BlockSpec((BLK, RNB), lambda i: (i, 0)),
            pl.BlockSpec((25 * cp, o), lambda i: (0, 0)),
            pl.BlockSpec((1, o), lambda i: (0, 0)),
        ],
        out_specs=[
            pl.BlockSpec((BLK, o), lambda i: (i, 0)),
            pl.BlockSpec((1, o), lambda i: (0, 0)),
            pl.BlockSpec((1, o), lambda i: (0, 0)),
        ],
        out_shape=[
            jax.ShapeDtypeStruct((NPAD, o), jnp.float32),
            jax.ShapeDtypeStruct((1, o), jnp.float32),
            jax.ShapeDtypeStruct((1, o), jnp.float32),
        ],
        scratch_shapes=[pltpu.VMEM((BLK, 25 * cp), jnp.float32)],
        interpret=interpret,
    )(g, w_tn, we, b)


def _bn_relu(y, sm, sq, gamma, beta, interpret=False):
    o = y.shape[1]
    nb = 2624
    nblk = NPAD // nb
    assert nblk * nb == NPAD

    def body(y_ref, sm_ref, sq_ref, g_ref, be_ref, h_ref):
        mean = sm_ref[0] * (1.0 / N_NODES)
        var = sq_ref[0] * (1.0 / N_NODES) - mean * mean
        s = g_ref[0] * lax.rsqrt(var + EPS)
        t = be_ref[0] - mean * s
        h = jnp.maximum(y_ref[:] * s + t, 0.0).astype(TDT)
        if o == 128:
            h_ref[:] = h
        else:
            h_ref[:, :o] = h
            h_ref[:, o:] = jnp.zeros((nb, 128 - o), TDT)

    return pl.pallas_call(
        body,
        grid=(nblk,),
        in_specs=[
            pl.BlockSpec((nb, o), lambda i: (i, 0)),
            pl.BlockSpec((1, o), lambda i: (0, 0)),
            pl.BlockSpec((1, o), lambda i: (0, 0)),
            pl.BlockSpec((1, o), lambda i: (0, 0)),
            pl.BlockSpec((1, o), lambda i: (0, 0)),
        ],
        out_specs=pl.BlockSpec((nb, 128), lambda i: (i, 0)),
        out_shape=jax.ShapeDtypeStruct((NPAD, 128), TDT),
        interpret=interpret,
    )(y, sm, sq, gamma, beta)


def _forward(x, neigh_indices, neigh_weights, W0, b0, gamma0, beta0, W1, b1,
             gamma1, beta1, W2, b2, gamma2, beta2, W_out, b_out,
             conv_interp_fn, interp0_fn, interpret=False):
    r2 = lambda v: v.reshape(1, -1)
    tapi, tapw = _tap_streams(neigh_indices, neigh_weights)

    xflat = jnp.pad(x.reshape(1, -1), ((0, 0), (0, XFL - 3 * N_NODES)))
    idx3, w3 = _l0_streams(neigh_indices, neigh_weights)
    interp0 = interp0_fn(xflat, idx3, w3)
    we0 = jnp.pad(W0.T, ((0, 53), (0, 0)))
    y0, sm0, sq0 = _matmul0(interp0, we0, r2(b0), 64, interpret)
    h0 = _bn_relu(y0, sm0, sq0, r2(gamma0), r2(beta0), interpret)

    i1 = conv_interp_fn(h0, tapi, tapw, 64).reshape(NPAD, 1600)
    y1, sm1, sq1 = _matmul0(i1, _expand_w(W1, 64, 64), r2(b1), 64, interpret)
    h1 = _bn_relu(y1, sm1, sq1, r2(gamma1), r2(beta1), interpret)

    i2 = conv_interp_fn(h1, tapi, tapw, 64).reshape(NPAD, 1600)
    y2, sm2, sq2 = _matmul0(i2, _expand_w(W2, 64, 64), r2(b2), 128, interpret)
    h2 = _bn_relu(y2, sm2, sq2, r2(gamma2), r2(beta2), interpret)

    i3 = conv_interp_fn(h2, tapi, tapw, 128).reshape(NPAD, 3200)
    y3, _, _ = _matmul0(i3, _expand_w(W_out, 128, 128), r2(b_out), 36,
                        interpret)
    return y3[:N_NODES]


def kernel(x, neigh_indices, neigh_weights, W0, b0, gamma0, beta0, W1, b1,
           gamma1, beta1, W2, b2, gamma2, beta2, W_out, b_out):
    return _forward(x, neigh_indices, neigh_weights, W0, b0, gamma0, beta0,
                    W1, b1, gamma1, beta1, W2, b2, gamma2, beta2, W_out,
                    b_out, _sc_conv_interp, _sc_interp0)


# speedup vs baseline: 1.0450x; 1.0450x over previous
import dataclasses
import functools

import jax
import jax.numpy as jnp
from jax import lax
from jax.experimental import pallas as pl
from jax.experimental.pallas import tpu as pltpu
from jax.experimental.pallas import tpu_sc as plsc

N_NODES = 10242
BLK = 256
NPAD = ((N_NODES + BLK - 1) // BLK) * BLK
RNB = 75
NUMIDX = RNB * NPAD
GW = 128
NGATH = NUMIDX // GW
EPS = 1e-5
NC, NS = 2, 16
NW = NC * NS
GPW = NGATH // NW
GREM = NGATH - GPW * NW
MAXG = -(-(GPW + GREM) // 8) * 8
NGPAD = (NW - 1) * GPW + MAXG
TDT = jnp.float32


def _sc_gather(table, idx2d):
    mesh = plsc.VectorSubcoreMesh(core_axis_name="c", subcore_axis_name="s")
    nbuf = 6

    @functools.partial(
        pl.kernel,
        mesh=mesh,
        out_type=jax.ShapeDtypeStruct((NUMIDX, 128), table.dtype),
        scratch_types=[pltpu.VMEM((MAXG, GW), jnp.int32)]
        + [pltpu.VMEM((GW, 128), table.dtype)] * nbuf
        + [pltpu.SemaphoreType.DMA] * nbuf,
    )
    def gk(table_hbm, idx_hbm, out_hbm, idx_s, *bufs_sems):
        bufs, sems = bufs_sems[:nbuf], bufs_sems[nbuf:]
        wid = lax.axis_index("s") * NC + lax.axis_index("c")
        g0 = wid * GPW
        ng = lax.select(wid == NW - 1, GPW + GREM, GPW)
        pltpu.sync_copy(idx_hbm.at[pl.ds(g0, MAXG)], idx_s)

        @pl.loop(0, ng // nbuf)
        def _(it):
            c = it * nbuf
            cps = [
                pltpu.async_copy(table_hbm.at[idx_s.at[c + b]], bufs[b],
                                 sems[b])
                for b in range(nbuf)
            ]
            for b in range(nbuf):
                cps[b].wait()
                pltpu.sync_copy(bufs[b],
                                out_hbm.at[pl.ds((g0 + c + b) * GW, GW)])

    return gk(table, idx2d)


XFL = 30728
NPW = NPAD // NW
L0C = 8


def _sc_interp0(xflat, comb):
    mesh = plsc.VectorSubcoreMesh(core_axis_name="c", subcore_axis_name="s")
    nch = NPW // L0C
    cp = pltpu.CompilerParams()
    if "needs_layout_passes" in pltpu.CompilerParams.__dataclass_fields__:
        cp = dataclasses.replace(cp, needs_layout_passes=False)

    @functools.partial(
        pl.kernel,
        mesh=mesh,
        compiler_params=cp,
        out_type=jax.ShapeDtypeStruct((NPAD, 128), jnp.float32),
        scratch_types=[
            pltpu.VMEM((1, XFL), jnp.float32),
            pltpu.VMEM((L0C, 512), jnp.int32),
            pltpu.VMEM((L0C, 512), jnp.int32),
            pltpu.VMEM((L0C, 128), jnp.float32),
            pltpu.SemaphoreType.DMA,
            pltpu.SemaphoreType.DMA,
        ],
    )
    def ik(x_hbm, s_hbm, o_hbm, x_v, s_a, s_b, o_v, sem_a, sem_b):
        wid = lax.axis_index("s") * NC + lax.axis_index("c")
        n0 = wid * NPW
        pltpu.sync_copy(x_hbm, x_v)

        def fetch(ci, buf, sem):
            pltpu.make_async_copy(s_hbm.at[pl.ds(n0 + ci * L0C, L0C)], buf,
                                  sem).start()

        def wait(ci, buf, sem):
            pltpu.make_async_copy(s_hbm.at[pl.ds(n0 + ci * L0C, L0C)], buf,
                                  sem).wait()

        def compute(ci, buf):
            for n in range(L0C):
                for v in range(5):
                    acc = None
                    for j in range(3):
                        o = j * 80 + v * 16
                        ivec = buf[n, pl.ds(o, 16)]
                        wvec = plsc.bitcast(buf[n, pl.ds(256 + o, 16)],
                                            jnp.float32)
                        term = wvec * plsc.load_gather(x_v.at[0], [ivec])
                        acc = term if acc is None else acc + term
                    o_v[n, pl.ds(v * 16, 16)] = acc
                for z in range(5, 8):
                    o_v[n, pl.ds(z * 16, 16)] = jnp.zeros((16,), jnp.float32)
            pltpu.sync_copy(o_v, o_hbm.at[pl.ds(n0 + ci * L0C, L0C)])

        fetch(0, s_a, sem_a)

        @pl.loop(0, (nch - 1) // 2)
        def _(p):
            ca = 2 * p
            fetch(ca + 1, s_b, sem_b)
            wait(ca, s_a, sem_a)
            compute(ca, s_a)
            fetch(ca + 2, s_a, sem_a)
            wait(ca + 1, s_b, sem_b)
            compute(ca + 1, s_b)

        wait(nch - 1, s_a, sem_a)
        compute(nch - 1, s_a)

    return ik(xflat, comb)


def _l0_streams(neigh_indices, neigh_weights):
    idxr = neigh_indices.reshape(N_NODES, 25, 3)
    nwr = neigh_weights
    cc = jnp.tile(jnp.arange(3, dtype=jnp.int32), 25)
    parts_i, parts_w = [], []
    for j in range(3):
        a = jnp.repeat(idxr[:, :, j], 3, axis=1) * 3 + cc
        w = jnp.repeat(nwr[:, :, j], 3, axis=1)
        parts_i.append(jnp.pad(a, ((0, 0), (0, 5))))
        parts_w.append(jnp.pad(w, ((0, 0), (0, 5))))
    idx3 = jnp.pad(jnp.concatenate(parts_i, axis=1),
                   ((0, NPAD - N_NODES), (0, 16)))
    w3 = jnp.pad(jnp.concatenate(parts_w, axis=1),
                 ((0, NPAD - N_NODES), (0, 16)))
    return jnp.concatenate(
        [idx3, jax.lax.bitcast_convert_type(w3, jnp.int32)], axis=1)


def _matmul0(interp0, we, b, o, interpret=False):
    nblk = NPAD // BLK
    kdim = interp0.shape[1]

    def body(a_ref, we_ref, b_ref, y_ref, sm_ref, sq_ref):
        i = pl.program_id(0)
        y = jnp.dot(a_ref[...], we_ref[...],
                    preferred_element_type=jnp.float32) + b_ref[0]
        y_ref[...] = y
        rid = i * BLK + lax.broadcasted_iota(jnp.int32, (BLK, 1), 0)
        ym = jnp.where(rid < N_NODES, y, 0.0)
        ps = jnp.sum(ym, axis=0, keepdims=True)
        ps2 = jnp.sum(ym * ym, axis=0, keepdims=True)

        @pl.when(i == 0)
        def _():
            sm_ref[...] = ps
            sq_ref[...] = ps2

        @pl.when(i > 0)
        def _():
            sm_ref[...] = sm_ref[...] + ps
            sq_ref[...] = sq_ref[...] + ps2

    return pl.pallas_call(
        body,
        grid=(nblk,),
        in_specs=[
            pl.BlockSpec((BLK, kdim), lambda i: (i, 0)),
            pl.BlockSpec((kdim, o), lambda i: (0, 0)),
            pl.BlockSpec((1, o), lambda i: (0, 0)),
        ],
        out_specs=[
            pl.BlockSpec((BLK, o), lambda i: (i, 0)),
            pl.BlockSpec((1, o), lambda i: (0, 0)),
            pl.BlockSpec((1, o), lambda i: (0, 0)),
        ],
        out_shape=[
            jax.ShapeDtypeStruct((NPAD, o), jnp.float32),
            jax.ShapeDtypeStruct((1, o), jnp.float32),
            jax.ShapeDtypeStruct((1, o), jnp.float32),
        ],
        interpret=interpret,
    )(interp0, we, b)


CHN = 4
NCH = NPAD // CHN
CHT = CHN * RNB
CHTP = CHT + 4
CPW = NCH // NW


def _sc_conv_interp(table, idxs, ws, cin):
    mesh = plsc.VectorSubcoreMesh(core_axis_name="c", subcore_axis_name="s")
    cp = pltpu.CompilerParams()
    if "needs_layout_passes" in pltpu.CompilerParams.__dataclass_fields__:
        cp = dataclasses.replace(cp, needs_layout_passes=False)
    kw = 25 * cin
    obuf = 8 * kw
    nv = cin // 16

    @functools.partial(
        pl.kernel,
        mesh=mesh,
        compiler_params=cp,
        out_type=jax.ShapeDtypeStruct((NPAD * kw,), jnp.float32),
        scratch_types=[
            pltpu.VMEM((CHTP,), jnp.int32),
            pltpu.VMEM((CHTP,), jnp.int32),
            pltpu.VMEM((CHTP, 128), jnp.float32),
            pltpu.VMEM((CHTP, 128), jnp.float32),
            pltpu.VMEM((CHTP * 16,), jnp.float32),
            pltpu.VMEM((obuf,), jnp.float32),
            pltpu.SemaphoreType.DMA,
            pltpu.SemaphoreType.DMA,
        ],
    )
    def ck(t_hbm, i_hbm, w_hbm, o_hbm, ia_v, ib_v, ta_v, tb_v, w_v, o_v,
           sa, sb):
        wid = lax.axis_index("s") * NC + lax.axis_index("c")
        c0 = wid * CPW

        def fetch(c, idx_v, taps_v, sem):
            pltpu.sync_copy(i_hbm.at[pl.ds(c * CHTP, CHTP)], idx_v)
            pltpu.make_async_copy(t_hbm.at[idx_v], taps_v, sem).start()

        def wait(idx_v, taps_v, sem):
            pltpu.make_async_copy(t_hbm.at[idx_v], taps_v, sem).wait()

        def compute(c, taps_v, half):
            pltpu.sync_copy(w_hbm.at[pl.ds(c * CHTP * 16, CHTP * 16)], w_v)

            @pl.loop(0, CHN)
            def _(n):
                rbase = n * RNB
                wbase = rbase * 16
                obase = (half * CHN) * kw + n * kw
                for k in range(25):
                    for v in range(nv):
                        acc = None
                        for j in range(3):
                            r = 3 * k + j
                            wv = w_v[pl.ds(wbase + r * 16, 16)]
                            term = wv * taps_v[rbase + r, pl.ds(16 * v, 16)]
                            acc = term if acc is None else acc + term
                        o_v[pl.ds(obase + k * cin + 16 * v, 16)] = acc

        fetch(c0, ia_v, ta_v, sa)

        @pl.loop(0, CPW // 2)
        def _(p):
            ca = c0 + 2 * p
            fetch(ca + 1, ib_v, tb_v, sb)
            wait(ia_v, ta_v, sa)
            compute(ca, ta_v, 0)
            fetch(ca + 2, ia_v, ta_v, sa)
            wait(ib_v, tb_v, sb)
            compute(ca + 1, tb_v, 1)
            pltpu.sync_copy(o_v, o_hbm.at[pl.ds((ca * CHN) * kw, obuf)])

        wait(ia_v, ta_v, sa)

    return ck(table, idxs, ws)


def _tap_streams(neigh_indices, neigh_weights):
    def prep(a, dt, rep):
        a = jnp.pad(a.reshape(N_NODES, RNB).astype(dt),
                    ((0, NPAD - N_NODES), (0, 0)))
        if rep > 1:
            a = jnp.repeat(a.reshape(-1, 1), rep, axis=1)
        a = jnp.pad(a.reshape(NCH, CHT * rep),
                    ((0, 8), (0, (CHTP - CHT) * rep)))
        return a.reshape(-1)

    return (prep(neigh_indices, jnp.int32, 1),
            prep(neigh_weights, jnp.float32, 16))


def _expand_w(w, ci, cp):
    o = w.shape[0]
    wr = w.reshape(o, 25, ci)
    if cp != ci:
        wr = jnp.pad(wr, ((0, 0), (0, 0), (0, cp - ci)))
    return wr.transpose(1, 2, 0).reshape(25 * cp, o)


def _conv(g, w_tn, we, b, o, cp, interpret=False):
    nblk = NPAD // BLK

    def body(g_ref, w_ref, we_ref, b_ref, y_ref, sm_ref, sq_ref, interp_ref):
        i = pl.program_id(0)
        for k in range(25):
            acc = None
            for j in range(3):
                r = 3 * k + j
                gr = g_ref[r, :, :cp].astype(jnp.float32)
                term = gr * w_ref[:, r : r + 1]
                acc = term if acc is None else acc + term
            interp_ref[:, k * cp : (k + 1) * cp] = acc
        y = jnp.dot(interp_ref[:], we_ref[:],
                    preferred_element_type=jnp.float32) + b_ref[0]
        y_ref[:] = y
        rid = i * BLK + lax.broadcasted_iota(jnp.int32, (BLK, 1), 0)
        ym = jnp.where(rid < N_NODES, y, 0.0)
        ps = jnp.sum(ym, axis=0, keepdims=True)
        ps2 = jnp.sum(ym * ym, axis=0, keepdims=True)

        @pl.when(i == 0)
        def _():
            sm_ref[:] = ps
            sq_ref[:] = ps2

        @pl.when(i > 0)
        def _():
            sm_ref[:] = sm_ref[:] + ps
            sq_ref[:] = sq_ref[:] + ps2

    return pl.pallas_call(
        body,
        grid=(nblk,),
        in_specs=[
            pl.BlockSpec((RNB, BLK, 128), lambda i: (0, i, 0)),
            pl.BlockSpec((BLK, RNB), lambda i: (i, 0)),
            pl.BlockSpec((25 * cp, o), lambda i: (0, 0)),
            pl.BlockSpec((1, o), lambda i: (0, 0)),
        ],
        out_specs=[
            pl.BlockSpec((BLK, o), lambda i: (i, 0)),
            pl.BlockSpec((1, o), lambda i: (0, 0)),
            pl.BlockSpec((1, o), lambda i: (0, 0)),
        ],
        out_shape=[
            jax.ShapeDtypeStruct((NPAD, o), jnp.float32),
            jax.ShapeDtypeStruct((1, o), jnp.float32),
            jax.ShapeDtypeStruct((1, o), jnp.float32),
        ],
        scratch_shapes=[pltpu.VMEM((BLK, 25 * cp), jnp.float32)],
        interpret=interpret,
    )(g, w_tn, we, b)


def _bn_relu(y, sm, sq, gamma, beta, interpret=False):
    o = y.shape[1]
    nb = 2624
    nblk = NPAD // nb
    assert nblk * nb == NPAD

    def body(y_ref, sm_ref, sq_ref, g_ref, be_ref, h_ref):
        mean = sm_ref[0] * (1.0 / N_NODES)
        var = sq_ref[0] * (1.0 / N_NODES) - mean * mean
        s = g_ref[0] * lax.rsqrt(var + EPS)
        t = be_ref[0] - mean * s
        h = jnp.maximum(y_ref[:] * s + t, 0.0).astype(TDT)
        if o == 128:
            h_ref[:] = h
        else:
            h_ref[:, :o] = h
            h_ref[:, o:] = jnp.zeros((nb, 128 - o), TDT)

    return pl.pallas_call(
        body,
        grid=(nblk,),
        in_specs=[
            pl.BlockSpec((nb, o), lambda i: (i, 0)),
            pl.BlockSpec((1, o), lambda i: (0, 0)),
            pl.BlockSpec((1, o), lambda i: (0, 0)),
            pl.BlockSpec((1, o), lambda i: (0, 0)),
            pl.BlockSpec((1, o), lambda i: (0, 0)),
        ],
        out_specs=pl.BlockSpec((nb, 128), lambda i: (i, 0)),
        out_shape=jax.ShapeDtypeStruct((NPAD, 128), TDT),
        interpret=interpret,
    )(y, sm, sq, gamma, beta)


def _forward(x, neigh_indices, neigh_weights, W0, b0, gamma0, beta0, W1, b1,
             gamma1, beta1, W2, b2, gamma2, beta2, W_out, b_out,
             conv_interp_fn, interp0_fn, interpret=False):
    r2 = lambda v: v.reshape(1, -1)
    tapi, tapw = _tap_streams(neigh_indices, neigh_weights)

    xflat = jnp.pad(x.reshape(1, -1), ((0, 0), (0, XFL - 3 * N_NODES)))
    comb0 = _l0_streams(neigh_indices, neigh_weights)
    interp0 = interp0_fn(xflat, comb0)
    we0 = jnp.pad(W0.T, ((0, 53), (0, 0)))
    y0, sm0, sq0 = _matmul0(interp0, we0, r2(b0), 64, interpret)
    h0 = _bn_relu(y0, sm0, sq0, r2(gamma0), r2(beta0), interpret)

    i1 = conv_interp_fn(h0, tapi, tapw, 64).reshape(NPAD, 1600)
    y1, sm1, sq1 = _matmul0(i1, _expand_w(W1, 64, 64), r2(b1), 64, interpret)
    h1 = _bn_relu(y1, sm1, sq1, r2(gamma1), r2(beta1), interpret)

    i2 = conv_interp_fn(h1, tapi, tapw, 64).reshape(NPAD, 1600)
    y2, sm2, sq2 = _matmul0(i2, _expand_w(W2, 64, 64), r2(b2), 128, interpret)
    h2 = _bn_relu(y2, sm2, sq2, r2(gamma2), r2(beta2), interpret)

    i3 = conv_interp_fn(h2, tapi, tapw, 128).reshape(NPAD, 3200)
    y3, _, _ = _matmul0(i3, _expand_w(W_out, 128, 128), r2(b_out), 36,
                        interpret)
    return y3[:N_NODES]


def kernel(x, neigh_indices, neigh_weights, W0, b0, gamma0, beta0, W1, b1,
           gamma1, beta1, W2, b2, gamma2, beta2, W_out, b_out):
    return _forward(x, neigh_indices, neigh_weights, W0, b0, gamma0, beta0,
                    W1, b1, gamma1, beta1, W2, b2, gamma2, beta2, W_out,
                    b_out, _sc_conv_interp, _sc_interp0)


# speedup vs baseline: 1.3079x; 1.2516x over previous
import dataclasses
import functools

import jax
import jax.numpy as jnp
from jax import lax
from jax.experimental import pallas as pl
from jax.experimental.pallas import tpu as pltpu
from jax.experimental.pallas import tpu_sc as plsc

N_NODES = 10242
BLK = 256
NPAD = ((N_NODES + BLK - 1) // BLK) * BLK
RNB = 75
NUMIDX = RNB * NPAD
GW = 128
NGATH = NUMIDX // GW
EPS = 1e-5
NC, NS = 2, 16
NW = NC * NS
GPW = NGATH // NW
GREM = NGATH - GPW * NW
MAXG = -(-(GPW + GREM) // 8) * 8
NGPAD = (NW - 1) * GPW + MAXG
TDT = jnp.float32


def _sc_gather(table, idx2d):
    mesh = plsc.VectorSubcoreMesh(core_axis_name="c", subcore_axis_name="s")
    nbuf = 6

    @functools.partial(
        pl.kernel,
        mesh=mesh,
        out_type=jax.ShapeDtypeStruct((NUMIDX, 128), table.dtype),
        scratch_types=[pltpu.VMEM((MAXG, GW), jnp.int32)]
        + [pltpu.VMEM((GW, 128), table.dtype)] * nbuf
        + [pltpu.SemaphoreType.DMA] * nbuf,
    )
    def gk(table_hbm, idx_hbm, out_hbm, idx_s, *bufs_sems):
        bufs, sems = bufs_sems[:nbuf], bufs_sems[nbuf:]
        wid = lax.axis_index("s") * NC + lax.axis_index("c")
        g0 = wid * GPW
        ng = lax.select(wid == NW - 1, GPW + GREM, GPW)
        pltpu.sync_copy(idx_hbm.at[pl.ds(g0, MAXG)], idx_s)

        @pl.loop(0, ng // nbuf)
        def _(it):
            c = it * nbuf
            cps = [
                pltpu.async_copy(table_hbm.at[idx_s.at[c + b]], bufs[b],
                                 sems[b])
                for b in range(nbuf)
            ]
            for b in range(nbuf):
                cps[b].wait()
                pltpu.sync_copy(bufs[b],
                                out_hbm.at[pl.ds((g0 + c + b) * GW, GW)])

    return gk(table, idx2d)


XFL = 30728
NPW = NPAD // NW
L0C = 8


def _sc_interp0(xflat, comb):
    mesh = plsc.VectorSubcoreMesh(core_axis_name="c", subcore_axis_name="s")
    nch = NPW // L0C
    cp = pltpu.CompilerParams()
    if "needs_layout_passes" in pltpu.CompilerParams.__dataclass_fields__:
        cp = dataclasses.replace(cp, needs_layout_passes=False)

    @functools.partial(
        pl.kernel,
        mesh=mesh,
        compiler_params=cp,
        out_type=jax.ShapeDtypeStruct((NPAD, 128), jnp.float32),
        scratch_types=[
            pltpu.VMEM((1, XFL), jnp.float32),
            pltpu.VMEM((L0C, 512), jnp.int32),
            pltpu.VMEM((L0C, 512), jnp.int32),
            pltpu.VMEM((L0C, 128), jnp.float32),
            pltpu.SemaphoreType.DMA,
            pltpu.SemaphoreType.DMA,
        ],
    )
    def ik(x_hbm, s_hbm, o_hbm, x_v, s_a, s_b, o_v, sem_a, sem_b):
        wid = lax.axis_index("s") * NC + lax.axis_index("c")
        n0 = wid * NPW
        pltpu.sync_copy(x_hbm, x_v)

        def fetch(ci, buf, sem):
            pltpu.make_async_copy(s_hbm.at[pl.ds(n0 + ci * L0C, L0C)], buf,
                                  sem).start()

        def wait(ci, buf, sem):
            pltpu.make_async_copy(s_hbm.at[pl.ds(n0 + ci * L0C, L0C)], buf,
                                  sem).wait()

        def compute(ci, buf):
            for n in range(L0C):
                for v in range(5):
                    acc = None
                    for j in range(3):
                        o = j * 80 + v * 16
                        ivec = buf[n, pl.ds(o, 16)]
                        wvec = plsc.bitcast(buf[n, pl.ds(256 + o, 16)],
                                            jnp.float32)
                        term = wvec * plsc.load_gather(x_v.at[0], [ivec])
                        acc = term if acc is None else acc + term
                    o_v[n, pl.ds(v * 16, 16)] = acc
                for z in range(5, 8):
                    o_v[n, pl.ds(z * 16, 16)] = jnp.zeros((16,), jnp.float32)
            pltpu.sync_copy(o_v, o_hbm.at[pl.ds(n0 + ci * L0C, L0C)])

        fetch(0, s_a, sem_a)

        @pl.loop(0, (nch - 1) // 2)
        def _(p):
            ca = 2 * p
            fetch(ca + 1, s_b, sem_b)
            wait(ca, s_a, sem_a)
            compute(ca, s_a)
            fetch(ca + 2, s_a, sem_a)
            wait(ca + 1, s_b, sem_b)
            compute(ca + 1, s_b)

        wait(nch - 1, s_a, sem_a)
        compute(nch - 1, s_a)

    return ik(xflat, comb)


def _l0_streams(neigh_indices, neigh_weights):
    idxr = neigh_indices.reshape(N_NODES, 25, 3)
    nwr = neigh_weights
    cc = jnp.tile(jnp.arange(3, dtype=jnp.int32), 25)
    parts_i, parts_w = [], []
    for j in range(3):
        a = jnp.repeat(idxr[:, :, j], 3, axis=1) * 3 + cc
        w = jnp.repeat(nwr[:, :, j], 3, axis=1)
        parts_i.append(jnp.pad(a, ((0, 0), (0, 5))))
        parts_w.append(jnp.pad(w, ((0, 0), (0, 5))))
    idx3 = jnp.pad(jnp.concatenate(parts_i, axis=1),
                   ((0, NPAD - N_NODES), (0, 16)))
    w3 = jnp.pad(jnp.concatenate(parts_w, axis=1),
                 ((0, NPAD - N_NODES), (0, 16)))
    return jnp.concatenate(
        [idx3, jax.lax.bitcast_convert_type(w3, jnp.int32)], axis=1)


def _matmul0(interp0, we, b, o, interpret=False):
    nblk = NPAD // BLK
    kdim = interp0.shape[1]

    def body(a_ref, we_ref, b_ref, y_ref, sm_ref, sq_ref):
        i = pl.program_id(0)
        y = jnp.dot(a_ref[...], we_ref[...],
                    preferred_element_type=jnp.float32) + b_ref[0]
        y_ref[...] = y
        rid = i * BLK + lax.broadcasted_iota(jnp.int32, (BLK, 1), 0)
        ym = jnp.where(rid < N_NODES, y, 0.0)
        ps = jnp.sum(ym, axis=0, keepdims=True)
        ps2 = jnp.sum(ym * ym, axis=0, keepdims=True)

        @pl.when(i == 0)
        def _():
            sm_ref[...] = ps
            sq_ref[...] = ps2

        @pl.when(i > 0)
        def _():
            sm_ref[...] = sm_ref[...] + ps
            sq_ref[...] = sq_ref[...] + ps2

    return pl.pallas_call(
        body,
        grid=(nblk,),
        in_specs=[
            pl.BlockSpec((BLK, kdim), lambda i: (i, 0)),
            pl.BlockSpec((kdim, o), lambda i: (0, 0)),
            pl.BlockSpec((1, o), lambda i: (0, 0)),
        ],
        out_specs=[
            pl.BlockSpec((BLK, o), lambda i: (i, 0)),
            pl.BlockSpec((1, o), lambda i: (0, 0)),
            pl.BlockSpec((1, o), lambda i: (0, 0)),
        ],
        out_shape=[
            jax.ShapeDtypeStruct((NPAD, o), jnp.float32),
            jax.ShapeDtypeStruct((1, o), jnp.float32),
            jax.ShapeDtypeStruct((1, o), jnp.float32),
        ],
        interpret=interpret,
    )(interp0, we, b)


CHN = 4
NCH = NPAD // CHN
CHT = CHN * RNB
CHTP = CHT + 4
CPW = NCH // NW


def _sc_conv_interp(table, idxs, ws, cin):
    mesh = plsc.VectorSubcoreMesh(core_axis_name="c", subcore_axis_name="s")
    cp = pltpu.CompilerParams()
    if "needs_layout_passes" in pltpu.CompilerParams.__dataclass_fields__:
        cp = dataclasses.replace(cp, needs_layout_passes=False)
    kw = 25 * cin
    obuf = 8 * kw
    nv = cin // 16

    @functools.partial(
        pl.kernel,
        mesh=mesh,
        compiler_params=cp,
        out_type=jax.ShapeDtypeStruct((NPAD * kw,), jnp.float32),
        scratch_types=[
            pltpu.VMEM((CHTP,), jnp.int32),
            pltpu.VMEM((CHTP,), jnp.int32),
            pltpu.VMEM((CHTP, 128), jnp.float32),
            pltpu.VMEM((CHTP, 128), jnp.float32),
            pltpu.VMEM((CHTP * 16,), jnp.float32),
            pltpu.VMEM((CHTP * 16,), jnp.float32),
            pltpu.VMEM((obuf,), jnp.float32),
            pltpu.SemaphoreType.DMA,
            pltpu.SemaphoreType.DMA,
            pltpu.SemaphoreType.DMA,
            pltpu.SemaphoreType.DMA,
        ],
    )
    def ck(t_hbm, i_hbm, w_hbm, o_hbm, ia_v, ib_v, ta_v, tb_v, wa_v, wb_v,
           o_v, sa, sb, swa, swb):
        wid = lax.axis_index("s") * NC + lax.axis_index("c")
        c0 = wid * CPW

        def fetch(c, idx_v, taps_v, w_v, sem, semw):
            pltpu.sync_copy(i_hbm.at[pl.ds(c * CHTP, CHTP)], idx_v)
            pltpu.make_async_copy(
                w_hbm.at[pl.ds(c * CHTP * 16, CHTP * 16)], w_v, semw).start()
            pltpu.make_async_copy(t_hbm.at[idx_v], taps_v, sem).start()

        def wait(c, idx_v, taps_v, w_v, sem, semw):
            pltpu.make_async_copy(
                w_hbm.at[pl.ds(c * CHTP * 16, CHTP * 16)], w_v, semw).wait()
            pltpu.make_async_copy(t_hbm.at[idx_v], taps_v, sem).wait()

        def compute(c, taps_v, w_v, half):
            @pl.loop(0, CHN)
            def _(n):
                rbase = n * RNB
                wbase = rbase * 16
                obase = (half * CHN) * kw + n * kw
                for k in range(25):
                    for v in range(nv):
                        acc = None
                        for j in range(3):
                            r = 3 * k + j
                            wv = w_v[pl.ds(wbase + r * 16, 16)]
                            term = wv * taps_v[rbase + r, pl.ds(16 * v, 16)]
                            acc = term if acc is None else acc + term
                        o_v[pl.ds(obase + k * cin + 16 * v, 16)] = acc

        fetch(c0, ia_v, ta_v, wa_v, sa, swa)

        @pl.loop(0, CPW // 2)
        def _(p):
            ca = c0 + 2 * p
            fetch(ca + 1, ib_v, tb_v, wb_v, sb, swb)
            wait(ca, ia_v, ta_v, wa_v, sa, swa)
            compute(ca, ta_v, wa_v, 0)
            fetch(ca + 2, ia_v, ta_v, wa_v, sa, swa)
            wait(ca + 1, ib_v, tb_v, wb_v, sb, swb)
            compute(ca + 1, tb_v, wb_v, 1)
            pltpu.sync_copy(o_v, o_hbm.at[pl.ds((ca * CHN) * kw, obuf)])

        wait(c0 + CPW, ia_v, ta_v, wa_v, sa, swa)

    return ck(table, idxs, ws)


def _tap_streams(neigh_indices, neigh_weights):
    def prep(a, dt, rep):
        a = jnp.pad(a.reshape(N_NODES, RNB).astype(dt),
                    ((0, NPAD - N_NODES), (0, 0)))
        if rep > 1:
            a = jnp.repeat(a.reshape(-1, 1), rep, axis=1)
        a = jnp.pad(a.reshape(NCH, CHT * rep),
                    ((0, 8), (0, (CHTP - CHT) * rep)))
        return a.reshape(-1)

    return (prep(neigh_indices, jnp.int32, 1),
            prep(neigh_weights, jnp.float32, 16))


def _expand_w(w, ci, cp):
    o = w.shape[0]
    wr = w.reshape(o, 25, ci)
    if cp != ci:
        wr = jnp.pad(wr, ((0, 0), (0, 0), (0, cp - ci)))
    return wr.transpose(1, 2, 0).reshape(25 * cp, o)


def _conv(g, w_tn, we, b, o, cp, interpret=False):
    nblk = NPAD // BLK

    def body(g_ref, w_ref, we_ref, b_ref, y_ref, sm_ref, sq_ref, interp_ref):
        i = pl.program_id(0)
        for k in range(25):
            acc = None
            for j in range(3):
                r = 3 * k + j
                gr = g_ref[r, :, :cp].astype(jnp.float32)
                term = gr * w_ref[:, r : r + 1]
                acc = term if acc is None else acc + term
            interp_ref[:, k * cp : (k + 1) * cp] = acc
        y = jnp.dot(interp_ref[:], we_ref[:],
                    preferred_element_type=jnp.float32) + b_ref[0]
        y_ref[:] = y
        rid = i * BLK + lax.broadcasted_iota(jnp.int32, (BLK, 1), 0)
        ym = jnp.where(rid < N_NODES, y, 0.0)
        ps = jnp.sum(ym, axis=0, keepdims=True)
        ps2 = jnp.sum(ym * ym, axis=0, keepdims=True)

        @pl.when(i == 0)
        def _():
            sm_ref[:] = ps
            sq_ref[:] = ps2

        @pl.when(i > 0)
        def _():
            sm_ref[:] = sm_ref[:] + ps
            sq_ref[:] = sq_ref[:] + ps2

    return pl.pallas_call(
        body,
        grid=(nblk,),
        in_specs=[
            pl.BlockSpec((RNB, BLK, 128), lambda i: (0, i, 0)),
            pl.BlockSpec((BLK, RNB), lambda i: (i, 0)),
            pl.BlockSpec((25 * cp, o), lambda i: (0, 0)),
            pl.BlockSpec((1, o), lambda i: (0, 0)),
        ],
        out_specs=[
            pl.BlockSpec((BLK, o), lambda i: (i, 0)),
            pl.BlockSpec((1, o), lambda i: (0, 0)),
            pl.BlockSpec((1, o), lambda i: (0, 0)),
        ],
        out_shape=[
            jax.ShapeDtypeStruct((NPAD, o), jnp.float32),
            jax.ShapeDtypeStruct((1, o), jnp.float32),
            jax.ShapeDtypeStruct((1, o), jnp.float32),
        ],
        scratch_shapes=[pltpu.VMEM((BLK, 25 * cp), jnp.float32)],
        interpret=interpret,
    )(g, w_tn, we, b)


def _bn_relu(y, sm, sq, gamma, beta, interpret=False):
    o = y.shape[1]
    nb = 2624
    nblk = NPAD // nb
    assert nblk * nb == NPAD

    def body(y_ref, sm_ref, sq_ref, g_ref, be_ref, h_ref):
        mean = sm_ref[0] * (1.0 / N_NODES)
        var = sq_ref[0] * (1.0 / N_NODES) - mean * mean
        s = g_ref[0] * lax.rsqrt(var + EPS)
        t = be_ref[0] - mean * s
        h = jnp.maximum(y_ref[:] * s + t, 0.0).astype(TDT)
        if o == 128:
            h_ref[:] = h
        else:
            h_ref[:, :o] = h
            h_ref[:, o:] = jnp.zeros((nb, 128 - o), TDT)

    return pl.pallas_call(
        body,
        grid=(nblk,),
        in_specs=[
            pl.BlockSpec((nb, o), lambda i: (i, 0)),
            pl.BlockSpec((1, o), lambda i: (0, 0)),
            pl.BlockSpec((1, o), lambda i: (0, 0)),
            pl.BlockSpec((1, o), lambda i: (0, 0)),
            pl.BlockSpec((1, o), lambda i: (0, 0)),
        ],
        out_specs=pl.BlockSpec((nb, 128), lambda i: (i, 0)),
        out_shape=jax.ShapeDtypeStruct((NPAD, 128), TDT),
        interpret=interpret,
    )(y, sm, sq, gamma, beta)


def _forward(x, neigh_indices, neigh_weights, W0, b0, gamma0, beta0, W1, b1,
             gamma1, beta1, W2, b2, gamma2, beta2, W_out, b_out,
             gather_fn, interp0_fn, interpret=False):
    r2 = lambda v: v.reshape(1, -1)
    idx_t = jnp.pad(neigh_indices.reshape(N_NODES, RNB).T,
                    ((0, 0), (0, NPAD - N_NODES)))
    idx2d = jnp.pad(idx_t.reshape(NGATH, GW), ((0, NGPAD - NGATH), (0, 0)))
    w_tn = jnp.pad(neigh_weights.reshape(N_NODES, RNB),
                   ((0, NPAD - N_NODES), (0, 0)))

    xflat = jnp.pad(x.reshape(1, -1), ((0, 0), (0, XFL - 3 * N_NODES)))
    comb0 = _l0_streams(neigh_indices, neigh_weights)
    interp0 = interp0_fn(xflat, comb0)
    we0 = jnp.pad(W0.T, ((0, 53), (0, 0)))
    y0, sm0, sq0 = _matmul0(interp0, we0, r2(b0), 64, interpret)
    h0 = _bn_relu(y0, sm0, sq0, r2(gamma0), r2(beta0), interpret)

    g1 = gather_fn(h0, idx2d).reshape(RNB, NPAD, 128)
    y1, sm1, sq1 = _conv(g1, w_tn, _expand_w(W1, 64, 64), r2(b1), 64, 64,
                         interpret)
    h1 = _bn_relu(y1, sm1, sq1, r2(gamma1), r2(beta1), interpret)

    g2 = gather_fn(h1, idx2d).reshape(RNB, NPAD, 128)
    y2, sm2, sq2 = _conv(g2, w_tn, _expand_w(W2, 64, 64), r2(b2), 128, 64,
                         interpret)
    h2 = _bn_relu(y2, sm2, sq2, r2(gamma2), r2(beta2), interpret)

    g3 = gather_fn(h2, idx2d).reshape(RNB, NPAD, 128)
    y3, _, _ = _conv(g3, w_tn, _expand_w(W_out, 128, 128), r2(b_out), 36,
                     128, interpret)
    return y3[:N_NODES]


def kernel(x, neigh_indices, neigh_weights, W0, b0, gamma0, beta0, W1, b1,
           gamma1, beta1, W2, b2, gamma2, beta2, W_out, b_out):
    return _forward(x, neigh_indices, neigh_weights, W0, b0, gamma0, beta0,
                    W1, b1, gamma1, beta1, W2, b2, gamma2, beta2, W_out,
                    b_out, _sc_gather, _sc_interp0)


# speedup vs baseline: 1.4088x; 1.0772x over previous
import dataclasses
import functools

import jax
import jax.numpy as jnp
from jax import lax
from jax.experimental import pallas as pl
from jax.experimental.pallas import tpu as pltpu
from jax.experimental.pallas import tpu_sc as plsc

N_NODES = 10242
BLK = 256
NPAD = ((N_NODES + BLK - 1) // BLK) * BLK
RNB = 75
NUMIDX = RNB * NPAD
GW = 128
NGATH = NUMIDX // GW
EPS = 1e-5
NC, NS = 2, 16
NW = NC * NS
GPW = NGATH // NW
GREM = NGATH - GPW * NW
MAXG = -(-(GPW + GREM) // 8) * 8
NGPAD = (NW - 1) * GPW + MAXG
TDT = jnp.float32


def _gather_geom(ngath, nbuf):
    gpw = ngath // NW
    grem = ngath - gpw * NW
    assert gpw % nbuf == 0 and (gpw + grem) % nbuf == 0
    maxg = -(-(gpw + grem + 7) // 8) * 8
    ngpad = ((NW - 1) * gpw // 8) * 8 + maxg
    return gpw, grem, maxg, ngpad


def _sc_gather(table, idx2d, ngath, nbuf):
    mesh = plsc.VectorSubcoreMesh(core_axis_name="c", subcore_axis_name="s")
    gpw, grem, maxg, _ = _gather_geom(ngath, nbuf)

    @functools.partial(
        pl.kernel,
        mesh=mesh,
        out_type=jax.ShapeDtypeStruct((ngath * GW, 128), table.dtype),
        scratch_types=[pltpu.VMEM((maxg, GW), jnp.int32)]
        + [pltpu.VMEM((GW, 128), table.dtype)] * nbuf
        + [pltpu.SemaphoreType.DMA] * nbuf,
    )
    def gk(table_hbm, idx_hbm, out_hbm, idx_s, *bufs_sems):
        bufs, sems = bufs_sems[:nbuf], bufs_sems[nbuf:]
        wid = lax.axis_index("s") * NC + lax.axis_index("c")
        g0 = wid * gpw
        base8 = (g0 // 8) * 8
        off = g0 - base8
        ng = lax.select(wid == NW - 1, gpw + grem, gpw)
        pltpu.sync_copy(idx_hbm.at[pl.ds(base8, maxg)], idx_s)

        @pl.loop(0, ng // nbuf)
        def _(it):
            c = it * nbuf
            cps = [
                pltpu.async_copy(table_hbm.at[idx_s.at[off + c + b]], bufs[b],
                                 sems[b])
                for b in range(nbuf)
            ]
            for b in range(nbuf):
                cps[b].wait()
                pltpu.sync_copy(bufs[b],
                                out_hbm.at[pl.ds((g0 + c + b) * GW, GW)])

    return gk(table, idx2d)


XFL = 30728
NPW = NPAD // NW
L0C = 8


def _sc_interp0(xflat, comb):
    mesh = plsc.VectorSubcoreMesh(core_axis_name="c", subcore_axis_name="s")
    nch = NPW // L0C
    cp = pltpu.CompilerParams()
    if "needs_layout_passes" in pltpu.CompilerParams.__dataclass_fields__:
        cp = dataclasses.replace(cp, needs_layout_passes=False)

    @functools.partial(
        pl.kernel,
        mesh=mesh,
        compiler_params=cp,
        out_type=jax.ShapeDtypeStruct((NPAD, 128), jnp.float32),
        scratch_types=[
            pltpu.VMEM((1, XFL), jnp.float32),
            pltpu.VMEM((L0C, 512), jnp.int32),
            pltpu.VMEM((L0C, 512), jnp.int32),
            pltpu.VMEM((L0C, 128), jnp.float32),
            pltpu.SemaphoreType.DMA,
            pltpu.SemaphoreType.DMA,
        ],
    )
    def ik(x_hbm, s_hbm, o_hbm, x_v, s_a, s_b, o_v, sem_a, sem_b):
        wid = lax.axis_index("s") * NC + lax.axis_index("c")
        n0 = wid * NPW
        pltpu.sync_copy(x_hbm, x_v)

        def fetch(ci, buf, sem):
            pltpu.make_async_copy(s_hbm.at[pl.ds(n0 + ci * L0C, L0C)], buf,
                                  sem).start()

        def wait(ci, buf, sem):
            pltpu.make_async_copy(s_hbm.at[pl.ds(n0 + ci * L0C, L0C)], buf,
                                  sem).wait()

        def compute(ci, buf):
            for n in range(L0C):
                for v in range(5):
                    acc = None
                    for j in range(3):
                        o = j * 80 + v * 16
                        ivec = buf[n, pl.ds(o, 16)]
                        wvec = plsc.bitcast(buf[n, pl.ds(256 + o, 16)],
                                            jnp.float32)
                        term = wvec * plsc.load_gather(x_v.at[0], [ivec])
                        acc = term if acc is None else acc + term
                    o_v[n, pl.ds(v * 16, 16)] = acc
                for z in range(5, 8):
                    o_v[n, pl.ds(z * 16, 16)] = jnp.zeros((16,), jnp.float32)
            pltpu.sync_copy(o_v, o_hbm.at[pl.ds(n0 + ci * L0C, L0C)])

        fetch(0, s_a, sem_a)

        @pl.loop(0, (nch - 1) // 2)
        def _(p):
            ca = 2 * p
            fetch(ca + 1, s_b, sem_b)
            wait(ca, s_a, sem_a)
            compute(ca, s_a)
            fetch(ca + 2, s_a, sem_a)
            wait(ca + 1, s_b, sem_b)
            compute(ca + 1, s_b)

        wait(nch - 1, s_a, sem_a)
        compute(nch - 1, s_a)

    return ik(xflat, comb)


def _l0_streams(neigh_indices, neigh_weights):
    idxr = neigh_indices.reshape(N_NODES, 25, 3)
    nwr = neigh_weights
    cc = jnp.tile(jnp.arange(3, dtype=jnp.int32), 25)
    parts_i, parts_w = [], []
    for j in range(3):
        a = jnp.repeat(idxr[:, :, j], 3, axis=1) * 3 + cc
        w = jnp.repeat(nwr[:, :, j], 3, axis=1)
        parts_i.append(jnp.pad(a, ((0, 0), (0, 5))))
        parts_w.append(jnp.pad(w, ((0, 0), (0, 5))))
    idx3 = jnp.pad(jnp.concatenate(parts_i, axis=1),
                   ((0, NPAD - N_NODES), (0, 16)))
    w3 = jnp.pad(jnp.concatenate(parts_w, axis=1),
                 ((0, NPAD - N_NODES), (0, 16)))
    return jnp.concatenate(
        [idx3, jax.lax.bitcast_convert_type(w3, jnp.int32)], axis=1)


def _matmul0(interp0, we, b, o, interpret=False):
    nblk = NPAD // BLK
    kdim = interp0.shape[1]

    def body(a_ref, we_ref, b_ref, y_ref, sm_ref, sq_ref):
        i = pl.program_id(0)
        y = jnp.dot(a_ref[...], we_ref[...],
                    preferred_element_type=jnp.float32) + b_ref[0]
        y_ref[...] = y
        rid = i * BLK + lax.broadcasted_iota(jnp.int32, (BLK, 1), 0)
        ym = jnp.where(rid < N_NODES, y, 0.0)
        ps = jnp.sum(ym, axis=0, keepdims=True)
        ps2 = jnp.sum(ym * ym, axis=0, keepdims=True)

        @pl.when(i == 0)
        def _():
            sm_ref[...] = ps
            sq_ref[...] = ps2

        @pl.when(i > 0)
        def _():
            sm_ref[...] = sm_ref[...] + ps
            sq_ref[...] = sq_ref[...] + ps2

    return pl.pallas_call(
        body,
        grid=(nblk,),
        in_specs=[
            pl.BlockSpec((BLK, kdim), lambda i: (i, 0)),
            pl.BlockSpec((kdim, o), lambda i: (0, 0)),
            pl.BlockSpec((1, o), lambda i: (0, 0)),
        ],
        out_specs=[
            pl.BlockSpec((BLK, o), lambda i: (i, 0)),
            pl.BlockSpec((1, o), lambda i: (0, 0)),
            pl.BlockSpec((1, o), lambda i: (0, 0)),
        ],
        out_shape=[
            jax.ShapeDtypeStruct((NPAD, o), jnp.float32),
            jax.ShapeDtypeStruct((1, o), jnp.float32),
            jax.ShapeDtypeStruct((1, o), jnp.float32),
        ],
        interpret=interpret,
    )(interp0, we, b)


CHN = 4
NCH = NPAD // CHN
CHT = CHN * RNB
CHTP = CHT + 4
CPW = NCH // NW


def _sc_conv_interp(table, idxs, ws, cin):
    mesh = plsc.VectorSubcoreMesh(core_axis_name="c", subcore_axis_name="s")
    cp = pltpu.CompilerParams()
    if "needs_layout_passes" in pltpu.CompilerParams.__dataclass_fields__:
        cp = dataclasses.replace(cp, needs_layout_passes=False)
    kw = 25 * cin
    obuf = 8 * kw
    nv = cin // 16

    @functools.partial(
        pl.kernel,
        mesh=mesh,
        compiler_params=cp,
        out_type=jax.ShapeDtypeStruct((NPAD * kw,), jnp.float32),
        scratch_types=[
            pltpu.VMEM((CHTP,), jnp.int32),
            pltpu.VMEM((CHTP,), jnp.int32),
            pltpu.VMEM((CHTP, 128), jnp.float32),
            pltpu.VMEM((CHTP, 128), jnp.float32),
            pltpu.VMEM((CHTP * 16,), jnp.float32),
            pltpu.VMEM((CHTP * 16,), jnp.float32),
            pltpu.VMEM((obuf,), jnp.float32),
            pltpu.SemaphoreType.DMA,
            pltpu.SemaphoreType.DMA,
            pltpu.SemaphoreType.DMA,
            pltpu.SemaphoreType.DMA,
        ],
    )
    def ck(t_hbm, i_hbm, w_hbm, o_hbm, ia_v, ib_v, ta_v, tb_v, wa_v, wb_v,
           o_v, sa, sb, swa, swb):
        wid = lax.axis_index("s") * NC + lax.axis_index("c")
        c0 = wid * CPW

        def fetch(c, idx_v, taps_v, w_v, sem, semw):
            pltpu.sync_copy(i_hbm.at[pl.ds(c * CHTP, CHTP)], idx_v)
            pltpu.make_async_copy(
                w_hbm.at[pl.ds(c * CHTP * 16, CHTP * 16)], w_v, semw).start()
            pltpu.make_async_copy(t_hbm.at[idx_v], taps_v, sem).start()

        def wait(c, idx_v, taps_v, w_v, sem, semw):
            pltpu.make_async_copy(
                w_hbm.at[pl.ds(c * CHTP * 16, CHTP * 16)], w_v, semw).wait()
            pltpu.make_async_copy(t_hbm.at[idx_v], taps_v, sem).wait()

        def compute(c, taps_v, w_v, half):
            @pl.loop(0, CHN)
            def _(n):
                rbase = n * RNB
                wbase = rbase * 16
                obase = (half * CHN) * kw + n * kw
                for k in range(25):
                    for v in range(nv):
                        acc = None
                        for j in range(3):
                            r = 3 * k + j
                            wv = w_v[pl.ds(wbase + r * 16, 16)]
                            term = wv * taps_v[rbase + r, pl.ds(16 * v, 16)]
                            acc = term if acc is None else acc + term
                        o_v[pl.ds(obase + k * cin + 16 * v, 16)] = acc

        fetch(c0, ia_v, ta_v, wa_v, sa, swa)

        @pl.loop(0, CPW // 2)
        def _(p):
            ca = c0 + 2 * p
            fetch(ca + 1, ib_v, tb_v, wb_v, sb, swb)
            wait(ca, ia_v, ta_v, wa_v, sa, swa)
            compute(ca, ta_v, wa_v, 0)
            fetch(ca + 2, ia_v, ta_v, wa_v, sa, swa)
            wait(ca + 1, ib_v, tb_v, wb_v, sb, swb)
            compute(ca + 1, tb_v, wb_v, 1)
            pltpu.sync_copy(o_v, o_hbm.at[pl.ds((ca * CHN) * kw, obuf)])

        wait(c0 + CPW, ia_v, ta_v, wa_v, sa, swa)

    return ck(table, idxs, ws)


def _tap_streams(neigh_indices, neigh_weights):
    def prep(a, dt, rep):
        a = jnp.pad(a.reshape(N_NODES, RNB).astype(dt),
                    ((0, NPAD - N_NODES), (0, 0)))
        if rep > 1:
            a = jnp.repeat(a.reshape(-1, 1), rep, axis=1)
        a = jnp.pad(a.reshape(NCH, CHT * rep),
                    ((0, 8), (0, (CHTP - CHT) * rep)))
        return a.reshape(-1)

    return (prep(neigh_indices, jnp.int32, 1),
            prep(neigh_weights, jnp.float32, 16))


def _expand_w(w, ci, cp):
    o = w.shape[0]
    wr = w.reshape(o, 25, ci)
    if cp != ci:
        wr = jnp.pad(wr, ((0, 0), (0, 0), (0, cp - ci)))
    return wr.transpose(1, 2, 0).reshape(25 * cp, o)


def _conv(g, w_tn, we, b, o, cp, nvalid, interpret=False):
    rows = g.shape[1]
    nblk = rows // BLK
    assert nblk * BLK == rows

    def body(g_ref, w_ref, we_ref, b_ref, y_ref, sm_ref, sq_ref, interp_ref):
        i = pl.program_id(0)
        for k in range(25):
            acc = None
            for j in range(3):
                r = 3 * k + j
                gr = g_ref[r, :, :cp].astype(jnp.float32)
                term = gr * w_ref[:, r : r + 1]
                acc = term if acc is None else acc + term
            interp_ref[:, k * cp : (k + 1) * cp] = acc
        y = jnp.dot(interp_ref[:], we_ref[:],
                    preferred_element_type=jnp.float32) + b_ref[0]
        y_ref[:] = y
        rid = i * BLK + lax.broadcasted_iota(jnp.int32, (BLK, 1), 0)
        ym = jnp.where(rid < nvalid, y, 0.0)
        ps = jnp.sum(ym, axis=0, keepdims=True)
        ps2 = jnp.sum(ym * ym, axis=0, keepdims=True)

        @pl.when(i == 0)
        def _():
            sm_ref[:] = ps
            sq_ref[:] = ps2

        @pl.when(i > 0)
        def _():
            sm_ref[:] = sm_ref[:] + ps
            sq_ref[:] = sq_ref[:] + ps2

    return pl.pallas_call(
        body,
        grid=(nblk,),
        in_specs=[
            pl.BlockSpec((RNB, BLK, 128), lambda i: (0, i, 0)),
            pl.BlockSpec((BLK, RNB), lambda i: (i, 0)),
            pl.BlockSpec((25 * cp, o), lambda i: (0, 0)),
            pl.BlockSpec((1, o), lambda i: (0, 0)),
        ],
        out_specs=[
            pl.BlockSpec((BLK, o), lambda i: (i, 0)),
            pl.BlockSpec((1, o), lambda i: (0, 0)),
            pl.BlockSpec((1, o), lambda i: (0, 0)),
        ],
        out_shape=[
            jax.ShapeDtypeStruct((rows, o), jnp.float32),
            jax.ShapeDtypeStruct((1, o), jnp.float32),
            jax.ShapeDtypeStruct((1, o), jnp.float32),
        ],
        scratch_shapes=[pltpu.VMEM((BLK, 25 * cp), jnp.float32)],
        interpret=interpret,
    )(g, w_tn, we, b)


def _bn_relu(y, sm, sq, gamma, beta, interpret=False):
    o = y.shape[1]
    rows = y.shape[0]
    nb = rows // 4
    nblk = 4
    assert nblk * nb == rows and nb % 8 == 0

    def body(y_ref, sm_ref, sq_ref, g_ref, be_ref, h_ref):
        mean = sm_ref[0] * (1.0 / N_NODES)
        var = sq_ref[0] * (1.0 / N_NODES) - mean * mean
        s = g_ref[0] * lax.rsqrt(var + EPS)
        t = be_ref[0] - mean * s
        h = jnp.maximum(y_ref[:] * s + t, 0.0).astype(TDT)
        if o == 128:
            h_ref[:] = h
        else:
            h_ref[:, :o] = h
            h_ref[:, o:] = jnp.zeros((nb, 128 - o), TDT)

    return pl.pallas_call(
        body,
        grid=(nblk,),
        in_specs=[
            pl.BlockSpec((nb, o), lambda i: (i, 0)),
            pl.BlockSpec((1, o), lambda i: (0, 0)),
            pl.BlockSpec((1, o), lambda i: (0, 0)),
            pl.BlockSpec((1, o), lambda i: (0, 0)),
            pl.BlockSpec((1, o), lambda i: (0, 0)),
        ],
        out_specs=pl.BlockSpec((nb, 128), lambda i: (i, 0)),
        out_shape=jax.ShapeDtypeStruct((rows, 128), TDT),
        interpret=interpret,
    )(y, sm, sq, gamma, beta)


def _forward(x, neigh_indices, neigh_weights, W0, b0, gamma0, beta0, W1, b1,
             gamma1, beta1, W2, b2, gamma2, beta2, W_out, b_out,
             gather_fn, interp0_fn, interpret=False):
    r2 = lambda v: v.reshape(1, -1)
    idx_t = jnp.pad(neigh_indices.reshape(N_NODES, RNB).T,
                    ((0, 0), (0, NPAD - N_NODES)))
    w_tf = jnp.pad(neigh_weights.reshape(N_NODES, RNB),
                   ((0, NPAD - N_NODES), (0, 0)))
    halves = []
    n0 = 0
    for rows, nbuf in ((5120, 3), (5376, 2)):
        ngath = RNB * rows // GW
        ngpad = _gather_geom(ngath, nbuf)[3]
        i2 = idx_t[:, n0:n0 + rows].reshape(ngath, GW)
        i2 = jnp.pad(i2, ((0, ngpad - ngath), (0, 0)))
        nvalid = min(rows, max(N_NODES - n0, 0))
        halves.append((rows, nbuf, ngath, i2, w_tf[n0:n0 + rows], nvalid))
        n0 += rows

    def conv_layer(h, we, b, o, cp):
        outs = []
        for rows, nbuf, ngath, i2, w_h, nvalid in halves:
            g = gather_fn(h, i2, ngath, nbuf).reshape(RNB, rows, 128)
            outs.append(_conv(g, w_h, we, b, o, cp, nvalid, interpret))
        (ya, sma, sqa), (yb, smb, sqb) = outs
        return ya, yb, sma + smb, sqa + sqb

    def bn_layer(ya, yb, sm, sq, gamma, beta):
        ha = _bn_relu(ya, sm, sq, gamma, beta, interpret)
        hb = _bn_relu(yb, sm, sq, gamma, beta, interpret)
        return jnp.concatenate([ha, hb], axis=0)

    xflat = jnp.pad(x.reshape(1, -1), ((0, 0), (0, XFL - 3 * N_NODES)))
    comb0 = _l0_streams(neigh_indices, neigh_weights)
    interp0 = interp0_fn(xflat, comb0)
    we0 = jnp.pad(W0.T, ((0, 53), (0, 0)))
    y0, sm0, sq0 = _matmul0(interp0, we0, r2(b0), 64, interpret)
    h0 = _bn_relu(y0, sm0, sq0, r2(gamma0), r2(beta0), interpret)

    y1a, y1b, sm1, sq1 = conv_layer(h0, _expand_w(W1, 64, 64), r2(b1), 64, 64)
    h1 = bn_layer(y1a, y1b, sm1, sq1, r2(gamma1), r2(beta1))

    y2a, y2b, sm2, sq2 = conv_layer(h1, _expand_w(W2, 64, 64), r2(b2), 128,
                                    64)
    h2 = bn_layer(y2a, y2b, sm2, sq2, r2(gamma2), r2(beta2))

    y3a, y3b, _, _ = conv_layer(h2, _expand_w(W_out, 128, 128), r2(b_out),
                                36, 128)
    return jnp.concatenate([y3a, y3b], axis=0)[:N_NODES]


def kernel(x, neigh_indices, neigh_weights, W0, b0, gamma0, beta0, W1, b1,
           gamma1, beta1, W2, b2, gamma2, beta2, W_out, b_out):
    return _forward(x, neigh_indices, neigh_weights, W0, b0, gamma0, beta0,
                    W1, b1, gamma1, beta1, W2, b2, gamma2, beta2, W_out,
                    b_out, _sc_gather, _sc_interp0)
